# Initial kernel scaffold; baseline (speedup 1.0000x reference)
import jax
import jax.numpy as jnp
from jax.experimental import pallas as pl


def kernel(feat):
    raise NotImplementedError("write your pallas kernel here")


# speedup vs baseline: 78.9332x; 78.9332x over previous
import jax
import jax.numpy as jnp
from jax.experimental import pallas as pl

_K = 64
_N = 12500
_D = 128
_ROW_CHUNK = 1024


def _row_maxes(feat_ref):
    parts = []
    c = 0
    while c < _N:
        w = min(_ROW_CHUNK, _N - c)
        parts.append(jnp.max(feat_ref[0, c : c + w, :], axis=-1))
        c += w
    return jnp.concatenate(parts)


def _bitonic_sort_rows(x):
    lane = jax.lax.broadcasted_iota(jnp.int32, (_K, _D), 1)
    k2 = 2
    while k2 <= _D:
        j = k2 // 2
        while j >= 1:
            rolled_p = jnp.concatenate([x[:, _D - j :], x[:, : _D - j]], axis=1)
            rolled_m = jnp.concatenate([x[:, j:], x[:, :j]], axis=1)
            partner = jnp.where((lane & j) != 0, rolled_p, rolled_m)
            take_min = ((lane & k2) == 0) == ((lane & j) == 0)
            x = jnp.where(take_min, jnp.minimum(x, partner),
                          jnp.maximum(x, partner))
            j //= 2
        k2 *= 2
    return x


def _sortpool_kernel(feat_ref, out_ref):
    v = _row_maxes(feat_ref)
    iota = jax.lax.iota(jnp.int32, _N)
    rows = []
    for _ in range(_K):
        m = jnp.max(v)
        idx = jnp.min(jnp.where(v == m, iota, _N))
        rows.append(feat_ref[0, pl.ds(idx, 1), :])
        v = jnp.where(iota == idx, -jnp.inf, v)
    sel = jnp.concatenate(rows, axis=0)
    out_ref[0] = _bitonic_sort_rows(sel)


def kernel(feat):
    b = feat.shape[0]
    pooled = pl.pallas_call(
        _sortpool_kernel,
        grid=(b,),
        in_specs=[pl.BlockSpec((1, _N, _D), lambda i: (i, 0, 0))],
        out_specs=pl.BlockSpec((1, _K, _D), lambda i: (i, 0, 0)),
        out_shape=jax.ShapeDtypeStruct((b, _K, _D), feat.dtype),
    )(feat)
    return pooled.reshape(b, _K * _D)


# speedup vs baseline: 202.8791x; 2.5703x over previous
import jax
import jax.numpy as jnp
from jax.experimental import pallas as pl
from jax.experimental.pallas import tpu as pltpu

_K = 64
_N = 12500
_D = 128
_NB = _N // _D
_TAIL = _N - _NB * _D
_MROWS = 128
_NEG = float("-inf")


def _roll_lanes(x, s):
    return jnp.concatenate([x[:, -s:], x[:, :-s]], axis=1)


def _kv_stage(keys, idx, j, take_min):
    lane = jax.lax.broadcasted_iota(jnp.int32, keys.shape, 1)
    bit = (lane & j) != 0
    kp = jnp.where(bit, _roll_lanes(keys, j), _roll_lanes(keys, -j))
    ip = jnp.where(bit, _roll_lanes(idx, j), _roll_lanes(idx, -j))
    self_hi = (keys > kp) | ((keys == kp) & (idx < ip))
    keep = self_hi != take_min
    return jnp.where(keep, keys, kp), jnp.where(keep, idx, ip)


def _sort_rows(keys, idx, desc_mask):
    lane = jax.lax.broadcasted_iota(jnp.int32, keys.shape, 1)
    k2 = 2
    while k2 <= _D:
        j = k2 // 2
        while j >= 1:
            take_min_asc = ((lane & k2) == 0) == ((lane & j) == 0)
            keys, idx = _kv_stage(keys, idx, j, take_min_asc != desc_mask)
            j //= 2
        k2 *= 2
    return keys, idx


def _merge_rows(keys, idx, desc_mask):
    lane = jax.lax.broadcasted_iota(jnp.int32, keys.shape, 1)
    j = _D // 2
    while j >= 1:
        keys, idx = _kv_stage(keys, idx, j, ((lane & j) == 0) != desc_mask)
        j //= 2
    return keys, idx


def _row_dir_mask(rows):
    if rows == 1:
        return jnp.full((1, _D), True)
    sub = jax.lax.broadcasted_iota(jnp.int32, (rows, _D), 0)
    return sub >= rows // 2


def _sort_rows_asc_plain(x):
    lane = jax.lax.broadcasted_iota(jnp.int32, x.shape, 1)
    k2 = 2
    while k2 <= _D:
        j = k2 // 2
        while j >= 1:
            bit = (lane & j) != 0
            p = jnp.where(bit, _roll_lanes(x, j), _roll_lanes(x, -j))
            take_min = ((lane & k2) == 0) == ((lane & j) == 0)
            x = jnp.where(take_min, jnp.minimum(x, p), jnp.maximum(x, p))
            j //= 2
        k2 *= 2
    return x


def _sortpool_kernel(feat_ref, out_ref, sel_ref):
    mrows = []
    for b in range(_NB):
        t = jnp.transpose(feat_ref[0, b * _D : (b + 1) * _D, :])
        mrows.append(jnp.max(t, axis=0, keepdims=True))
    tail = feat_ref[0, _NB * _D : _N, :]
    mt = jnp.transpose(jnp.max(tail, axis=1, keepdims=True))
    mrows.append(jnp.concatenate(
        [mt, jnp.full((1, _D - _TAIL), _NEG, jnp.float32)], axis=1))
    mrows.append(jnp.full((_MROWS - _NB - 1, _D), _NEG, jnp.float32))
    keys = jnp.concatenate(mrows, axis=0)
    sub = jax.lax.broadcasted_iota(jnp.int32, (_MROWS, _D), 0)
    lanei = jax.lax.broadcasted_iota(jnp.int32, (_MROWS, _D), 1)
    idx = sub * _D + lanei

    keys, idx = _sort_rows(keys, idx, _row_dir_mask(_MROWS))
    r = _MROWS
    while r > 1:
        h = r // 2
        ka, ia = keys[:h], idx[:h]
        kb, ib = keys[h:r], idx[h:r]
        self_hi = (ka > kb) | ((ka == kb) & (ia < ib))
        keys = jnp.where(self_hi, ka, kb)
        idx = jnp.where(self_hi, ia, ib)
        keys, idx = _merge_rows(keys, idx, _row_dir_mask(h))
        r = h

    for k in range(_K):
        s = idx[0, k]
        sel_ref[k : k + 1, :] = feat_ref[0, pl.ds(s, 1), :]
    out_ref[0] = _sort_rows_asc_plain(sel_ref[...])


def kernel(feat):
    b = feat.shape[0]
    pooled = pl.pallas_call(
        _sortpool_kernel,
        grid=(b,),
        in_specs=[pl.BlockSpec((1, _N, _D), lambda i: (i, 0, 0))],
        out_specs=pl.BlockSpec((1, _K, _D), lambda i: (i, 0, 0)),
        out_shape=jax.ShapeDtypeStruct((b, _K, _D), feat.dtype),
        scratch_shapes=[pltpu.VMEM((_K, _D), jnp.float32)],
    )(feat)
    return pooled.reshape(b, _K * _D)


# speedup vs baseline: 217.1242x; 1.0702x over previous
import jax
import jax.numpy as jnp
from jax.experimental import pallas as pl
from jax.experimental.pallas import tpu as pltpu

_K = 64
_N = 12500
_D = 128
_NB = _N // _D
_TAIL = _N - _NB * _D
_MROWS = 128
_BPB = 2
_NEG = float("-inf")


def _roll_lanes(x, s):
    return jnp.concatenate([x[:, -s:], x[:, :-s]], axis=1)


def _kv_stage(keys, idx, j, take_min):
    lane = jax.lax.broadcasted_iota(jnp.int32, keys.shape, 1)
    bit = (lane & j) != 0
    kp = jnp.where(bit, _roll_lanes(keys, j), _roll_lanes(keys, -j))
    ip = jnp.where(bit, _roll_lanes(idx, j), _roll_lanes(idx, -j))
    self_hi = (keys > kp) | ((keys == kp) & (idx < ip))
    keep = self_hi != take_min
    return jnp.where(keep, keys, kp), jnp.where(keep, idx, ip)


def _sort_rows(keys, idx, desc_mask):
    lane = jax.lax.broadcasted_iota(jnp.int32, keys.shape, 1)
    k2 = 2
    while k2 <= _D:
        j = k2 // 2
        while j >= 1:
            take_min_asc = ((lane & k2) == 0) == ((lane & j) == 0)
            keys, idx = _kv_stage(keys, idx, j, take_min_asc != desc_mask)
            j //= 2
        k2 *= 2
    return keys, idx


def _merge_rows(keys, idx, desc_mask):
    lane = jax.lax.broadcasted_iota(jnp.int32, keys.shape, 1)
    j = _D // 2
    while j >= 1:
        keys, idx = _kv_stage(keys, idx, j, ((lane & j) == 0) != desc_mask)
        j //= 2
    return keys, idx


def _row_dir_mask(rows):
    if rows == 1:
        return jnp.full((1, _D), True)
    sub = jax.lax.broadcasted_iota(jnp.int32, (rows, _D), 0)
    return sub >= rows // 2


def _sort_rows_asc_plain(x):
    lane = jax.lax.broadcasted_iota(jnp.int32, x.shape, 1)
    k2 = 2
    while k2 <= _D:
        j = k2 // 2
        while j >= 1:
            bit = (lane & j) != 0
            p = jnp.where(bit, _roll_lanes(x, j), _roll_lanes(x, -j))
            take_min = ((lane & k2) == 0) == ((lane & j) == 0)
            x = jnp.where(take_min, jnp.minimum(x, p), jnp.maximum(x, p))
            j //= 2
        k2 *= 2
    return x


def _sortpool_one_batch(feat_ref, out_ref, sel_ref, bb):
    mrows = []
    for b in range(_NB):
        t = jnp.transpose(feat_ref[bb, b * _D : (b + 1) * _D, :])
        mrows.append(jnp.max(t, axis=0, keepdims=True))
    tail = feat_ref[bb, _NB * _D : _N, :]
    mt = jnp.transpose(jnp.max(tail, axis=1, keepdims=True))
    mrows.append(jnp.concatenate(
        [mt, jnp.full((1, _D - _TAIL), _NEG, jnp.float32)], axis=1))
    mrows.append(jnp.full((_MROWS - _NB - 1, _D), _NEG, jnp.float32))
    keys = jnp.concatenate(mrows, axis=0)
    sub = jax.lax.broadcasted_iota(jnp.int32, (_MROWS, _D), 0)
    lanei = jax.lax.broadcasted_iota(jnp.int32, (_MROWS, _D), 1)
    idx = sub * _D + lanei

    keys, idx = _sort_rows(keys, idx, _row_dir_mask(_MROWS))
    r = _MROWS
    while r > 1:
        h = r // 2
        ka, ia = keys[:h], idx[:h]
        kb, ib = keys[h:r], idx[h:r]
        self_hi = (ka > kb) | ((ka == kb) & (ia < ib))
        keys = jnp.where(self_hi, ka, kb)
        idx = jnp.where(self_hi, ia, ib)
        keys, idx = _merge_rows(keys, idx, _row_dir_mask(h))
        r = h

    for k in range(_K):
        s = idx[0, k]
        sel_ref[bb, k : k + 1, :] = feat_ref[bb, pl.ds(s, 1), :]
    out_ref[bb] = _sort_rows_asc_plain(sel_ref[bb])


def _sortpool_kernel(feat_ref, out_ref, sel_ref):
    for bb in range(_BPB):
        _sortpool_one_batch(feat_ref, out_ref, sel_ref, bb)


def kernel(feat):
    b = feat.shape[0]
    pooled = pl.pallas_call(
        _sortpool_kernel,
        grid=(b // _BPB,),
        in_specs=[pl.BlockSpec((_BPB, _N, _D), lambda i: (i, 0, 0))],
        out_specs=pl.BlockSpec((_BPB, _K, _D), lambda i: (i, 0, 0)),
        out_shape=jax.ShapeDtypeStruct((b, _K, _D), feat.dtype),
        scratch_shapes=[pltpu.VMEM((_BPB, _K, _D), jnp.float32)],
    )(feat)
    return pooled.reshape(b, _K * _D)


# speedup vs baseline: 217.4984x; 1.0017x over previous
import jax
import jax.numpy as jnp
from jax.experimental import pallas as pl
from jax.experimental.pallas import tpu as pltpu

_K = 64
_N = 12500
_D = 128
_NB = _N // _D
_TAIL = _N - _NB * _D
_MROWS = 128
_BPB = 2
_NEG = float("-inf")


def _roll_lanes(x, s):
    return jnp.concatenate([x[:, -s:], x[:, :-s]], axis=1)


def _kv_stage(keys, idx, j, take_min):
    lane = jax.lax.broadcasted_iota(jnp.int32, keys.shape, 1)
    bit = (lane & j) != 0
    kp = jnp.where(bit, _roll_lanes(keys, j), _roll_lanes(keys, -j))
    ip = jnp.where(bit, _roll_lanes(idx, j), _roll_lanes(idx, -j))
    self_hi = (keys > kp) | ((keys == kp) & (idx < ip))
    keep = self_hi != take_min
    return jnp.where(keep, keys, kp), jnp.where(keep, idx, ip)


def _sort_rows(keys, idx, desc_mask):
    lane = jax.lax.broadcasted_iota(jnp.int32, keys.shape, 1)
    k2 = 2
    while k2 <= _D:
        j = k2 // 2
        while j >= 1:
            take_min_asc = ((lane & k2) == 0) == ((lane & j) == 0)
            keys, idx = _kv_stage(keys, idx, j, take_min_asc != desc_mask)
            j //= 2
        k2 *= 2
    return keys, idx


def _merge_rows(keys, idx, desc_mask):
    lane = jax.lax.broadcasted_iota(jnp.int32, keys.shape, 1)
    j = _D // 2
    while j >= 1:
        keys, idx = _kv_stage(keys, idx, j, ((lane & j) == 0) != desc_mask)
        j //= 2
    return keys, idx


def _row_dir_mask(rows):
    if rows == 1:
        return jnp.full((1, _D), True)
    sub = jax.lax.broadcasted_iota(jnp.int32, (rows, _D), 0)
    return sub >= rows // 2


def _sort_rows_asc_plain(x):
    lane = jax.lax.broadcasted_iota(jnp.int32, x.shape, 1)
    k2 = 2
    while k2 <= _D:
        j = k2 // 2
        while j >= 1:
            bit = (lane & j) != 0
            p = jnp.where(bit, _roll_lanes(x, j), _roll_lanes(x, -j))
            take_min = ((lane & k2) == 0) == ((lane & j) == 0)
            x = jnp.where(take_min, jnp.minimum(x, p), jnp.maximum(x, p))
            j //= 2
        k2 *= 2
    return x


def _sortpool_one_batch(feat_ref, out_ref, sel_ref, bb):
    mrows = []
    for b in range(_NB):
        t = jnp.transpose(feat_ref[bb, b * _D : (b + 1) * _D, :])
        mrows.append(jnp.max(t, axis=0, keepdims=True))
    tail = feat_ref[bb, _NB * _D : _N, :]
    mt = jnp.transpose(jnp.max(tail, axis=1, keepdims=True))
    mrows.append(jnp.concatenate(
        [mt, jnp.full((1, _D - _TAIL), _NEG, jnp.float32)], axis=1))
    mrows.append(jnp.full((_MROWS - _NB - 1, _D), _NEG, jnp.float32))
    keys = jnp.concatenate(mrows, axis=0)
    sub = jax.lax.broadcasted_iota(jnp.int32, (_MROWS, _D), 0)
    lanei = jax.lax.broadcasted_iota(jnp.int32, (_MROWS, _D), 1)
    idx = sub * _D + lanei

    keys, idx = _sort_rows(keys, idx, _row_dir_mask(_MROWS))
    r = _MROWS
    while r > 1:
        h = r // 2
        ka, ia = keys[:h], idx[:h]
        kb, ib = keys[h:r], idx[h:r]
        self_hi = (ka > kb) | ((ka == kb) & (ia < ib))
        keys = jnp.where(self_hi, ka, kb)
        idx = jnp.where(self_hi, ia, ib)
        keys, idx = _merge_rows(keys, idx, _row_dir_mask(h))
        r = h

    for k in range(_K):
        s = idx[0, k]
        sel_ref[bb, k : k + 1, :] = feat_ref[bb, pl.ds(s, 1), :]
    out_ref[bb] = _sort_rows_asc_plain(sel_ref[bb])


def _sortpool_kernel(hbm_ref, out_ref, buf_ref, sel_ref, sem_ref):
    i = pl.program_id(0)
    par = jax.lax.rem(i, 2)

    def copy(step, slot):
        return pltpu.make_async_copy(
            hbm_ref.at[pl.ds(step * _BPB, _BPB)],
            buf_ref.at[slot],
            sem_ref.at[slot],
        )

    @pl.when(i == 0)
    def _():
        copy(0, 0).start()

    copy(i, par).wait()

    @pl.when(i + 1 < pl.num_programs(0))
    def _():
        copy(i + 1, 1 - par).start()

    feat_view = buf_ref.at[par]
    for bb in range(_BPB):
        _sortpool_one_batch(feat_view, out_ref, sel_ref, bb)


def kernel(feat):
    b = feat.shape[0]
    pooled = pl.pallas_call(
        _sortpool_kernel,
        grid=(b // _BPB,),
        in_specs=[pl.BlockSpec(memory_space=pl.ANY)],
        out_specs=pl.BlockSpec((_BPB, _K, _D), lambda i: (i, 0, 0)),
        out_shape=jax.ShapeDtypeStruct((b, _K, _D), feat.dtype),
        scratch_shapes=[
            pltpu.VMEM((2, _BPB, _N, _D), jnp.float32),
            pltpu.VMEM((_BPB, _K, _D), jnp.float32),
            pltpu.SemaphoreType.DMA((2,)),
        ],
    )(feat)
    return pooled.reshape(b, _K * _D)


# speedup vs baseline: 244.5944x; 1.1246x over previous
import jax
import jax.numpy as jnp
from jax.experimental import pallas as pl
from jax.experimental.pallas import tpu as pltpu

_K = 64
_N = 12500
_D = 128
_NB = _N // _D
_TAIL = _N - _NB * _D
_MROWS = 128
_BPB = 2
_NEG = float("-inf")


def _roll_lanes(x, s):
    return jnp.concatenate([x[..., -s:], x[..., :-s]], axis=-1)


def _lane_iota(shape):
    return jax.lax.broadcasted_iota(jnp.int32, shape, len(shape) - 1)


def _kv_stage(keys, idx, j, take_min):
    bit = (_lane_iota(keys.shape) & j) != 0
    kp = jnp.where(bit, _roll_lanes(keys, j), _roll_lanes(keys, -j))
    ip = jnp.where(bit, _roll_lanes(idx, j), _roll_lanes(idx, -j))
    self_hi = (keys > kp) | ((keys == kp) & (idx < ip))
    keep = self_hi != take_min
    return jnp.where(keep, keys, kp), jnp.where(keep, idx, ip)


def _sort_rows(keys, idx, desc_mask):
    lane = _lane_iota(keys.shape)
    k2 = 2
    while k2 <= _D:
        j = k2 // 2
        while j >= 1:
            take_min_asc = ((lane & k2) == 0) == ((lane & j) == 0)
            keys, idx = _kv_stage(keys, idx, j, take_min_asc != desc_mask)
            j //= 2
        k2 *= 2
    return keys, idx


def _merge_rows(keys, idx, desc_mask):
    lane = _lane_iota(keys.shape)
    j = _D // 2
    while j >= 1:
        keys, idx = _kv_stage(keys, idx, j, ((lane & j) == 0) != desc_mask)
        j //= 2
    return keys, idx


def _row_dir_mask(rows):
    if rows == 1:
        return jnp.full((1, _D), True)
    sub = jax.lax.broadcasted_iota(jnp.int32, (rows, _D), 0)
    return sub >= rows // 2


def _sort_rows_asc_plain(x):
    lane = _lane_iota(x.shape)
    k2 = 2
    while k2 <= _D:
        j = k2 // 2
        while j >= 1:
            bit = (lane & j) != 0
            p = jnp.where(bit, _roll_lanes(x, j), _roll_lanes(x, -j))
            take_min = ((lane & k2) == 0) == ((lane & j) == 0)
            x = jnp.where(take_min, jnp.minimum(x, p), jnp.maximum(x, p))
            j //= 2
        k2 *= 2
    return x


def _batch_keys(feat_ref, bb):
    mrows = []
    for b in range(_NB):
        t = jnp.transpose(feat_ref[bb, b * _D : (b + 1) * _D, :])
        mrows.append(jnp.max(t, axis=0, keepdims=True))
    tail = feat_ref[bb, _NB * _D : _N, :]
    mt = jnp.transpose(jnp.max(tail, axis=1, keepdims=True))
    mrows.append(jnp.concatenate(
        [mt, jnp.full((1, _D - _TAIL), _NEG, jnp.float32)], axis=1))
    mrows.append(jnp.full((_MROWS - _NB - 1, _D), _NEG, jnp.float32))
    return jnp.concatenate(mrows, axis=0)


def _sortpool_kernel(feat_ref, out_ref, sel_ref):
    keys = jnp.stack([_batch_keys(feat_ref, bb) for bb in range(_BPB)])
    sub = jax.lax.broadcasted_iota(jnp.int32, keys.shape, 1)
    idx = sub * _D + _lane_iota(keys.shape)

    keys, idx = _sort_rows(keys, idx, _row_dir_mask(_MROWS))
    r = _MROWS
    while r > 1:
        h = r // 2
        ka, ia = keys[:, :h], idx[:, :h]
        kb, ib = keys[:, h:r], idx[:, h:r]
        self_hi = (ka > kb) | ((ka == kb) & (ia < ib))
        keys = jnp.where(self_hi, ka, kb)
        idx = jnp.where(self_hi, ia, ib)
        keys, idx = _merge_rows(keys, idx, _row_dir_mask(h))
        r = h

    for k in range(_K):
        for bb in range(_BPB):
            s = idx[bb, 0, k]
            sel_ref[bb, k : k + 1, :] = feat_ref[bb, pl.ds(s, 1), :]
    out_ref[...] = _sort_rows_asc_plain(sel_ref[...])


def kernel(feat):
    b = feat.shape[0]
    pooled = pl.pallas_call(
        _sortpool_kernel,
        grid=(b // _BPB,),
        in_specs=[pl.BlockSpec((_BPB, _N, _D), lambda i: (i, 0, 0))],
        out_specs=pl.BlockSpec((_BPB, _K, _D), lambda i: (i, 0, 0)),
        out_shape=jax.ShapeDtypeStruct((b, _K, _D), feat.dtype),
        scratch_shapes=[pltpu.VMEM((_BPB, _K, _D), jnp.float32)],
    )(feat)
    return pooled.reshape(b, _K * _D)


# speedup vs baseline: 264.3867x; 1.0809x over previous
import jax
import jax.numpy as jnp
from jax.experimental import pallas as pl
from jax.experimental.pallas import tpu as pltpu

_K = 64
_N = 12500
_D = 128
_NB = _N // _D
_TAIL = _N - _NB * _D
_MROWS = 128
_BPB = 4
_NEG = float("-inf")


def _roll_lanes(x, s):
    return jnp.concatenate([x[..., -s:], x[..., :-s]], axis=-1)


def _lane_iota(shape):
    return jax.lax.broadcasted_iota(jnp.int32, shape, len(shape) - 1)


def _kv_stage(keys, idx, j, take_min):
    bit = (_lane_iota(keys.shape) & j) != 0
    kp = jnp.where(bit, _roll_lanes(keys, j), _roll_lanes(keys, -j))
    ip = jnp.where(bit, _roll_lanes(idx, j), _roll_lanes(idx, -j))
    self_hi = (keys > kp) | ((keys == kp) & (idx < ip))
    keep = self_hi != take_min
    return jnp.where(keep, keys, kp), jnp.where(keep, idx, ip)


def _sort_rows(keys, idx, desc_mask):
    lane = _lane_iota(keys.shape)
    k2 = 2
    while k2 <= _D:
        j = k2 // 2
        while j >= 1:
            take_min_asc = ((lane & k2) == 0) == ((lane & j) == 0)
            keys, idx = _kv_stage(keys, idx, j, take_min_asc != desc_mask)
            j //= 2
        k2 *= 2
    return keys, idx


def _merge_rows(keys, idx, desc_mask):
    lane = _lane_iota(keys.shape)
    j = _D // 2
    while j >= 1:
        keys, idx = _kv_stage(keys, idx, j, ((lane & j) == 0) != desc_mask)
        j //= 2
    return keys, idx


def _row_dir_mask(rows):
    if rows == 1:
        return jnp.full((1, _D), True)
    sub = jax.lax.broadcasted_iota(jnp.int32, (rows, _D), 0)
    return sub >= rows // 2


def _sort_rows_asc_plain(x):
    lane = _lane_iota(x.shape)
    k2 = 2
    while k2 <= _D:
        j = k2 // 2
        while j >= 1:
            bit = (lane & j) != 0
            p = jnp.where(bit, _roll_lanes(x, j), _roll_lanes(x, -j))
            take_min = ((lane & k2) == 0) == ((lane & j) == 0)
            x = jnp.where(take_min, jnp.minimum(x, p), jnp.maximum(x, p))
            j //= 2
        k2 *= 2
    return x


def _batch_keys(feat_ref, bb):
    mrows = []
    for b in range(_NB):
        t = jnp.transpose(feat_ref[bb, b * _D : (b + 1) * _D, :])
        mrows.append(jnp.max(t, axis=0, keepdims=True))
    tail = feat_ref[bb, _NB * _D : _N, :]
    mt = jnp.transpose(jnp.max(tail, axis=1, keepdims=True))
    mrows.append(jnp.concatenate(
        [mt, jnp.full((1, _D - _TAIL), _NEG, jnp.float32)], axis=1))
    mrows.append(jnp.full((_MROWS - _NB - 1, _D), _NEG, jnp.float32))
    return jnp.concatenate(mrows, axis=0)


def _sortpool_kernel(feat_ref, out_ref, sel_ref):
    keys = jnp.stack([_batch_keys(feat_ref, bb) for bb in range(_BPB)])
    sub = jax.lax.broadcasted_iota(jnp.int32, keys.shape, 1)
    idx = sub * _D + _lane_iota(keys.shape)

    keys, idx = _sort_rows(keys, idx, _row_dir_mask(_MROWS))
    r = _MROWS
    while r > 1:
        h = r // 2
        ka, ia = keys[:, :h], idx[:, :h]
        kb, ib = keys[:, h:r], idx[:, h:r]
        self_hi = (ka > kb) | ((ka == kb) & (ia < ib))
        keys = jnp.where(self_hi, ka, kb)
        idx = jnp.where(self_hi, ia, ib)
        keys, idx = _merge_rows(keys, idx, _row_dir_mask(h))
        r = h

    for k in range(_K):
        for bb in range(_BPB):
            s = idx[bb, 0, k]
            sel_ref[bb, k : k + 1, :] = feat_ref[bb, pl.ds(s, 1), :]
    out_ref[...] = _sort_rows_asc_plain(sel_ref[...])


def kernel(feat):
    b = feat.shape[0]
    pooled = pl.pallas_call(
        _sortpool_kernel,
        grid=(b // _BPB,),
        in_specs=[pl.BlockSpec((_BPB, _N, _D), lambda i: (i, 0, 0))],
        out_specs=pl.BlockSpec((_BPB, _K, _D), lambda i: (i, 0, 0)),
        out_shape=jax.ShapeDtypeStruct((b, _K, _D), feat.dtype),
        scratch_shapes=[pltpu.VMEM((_BPB, _K, _D), jnp.float32)],
    )(feat)
    return pooled.reshape(b, _K * _D)


# speedup vs baseline: 267.5665x; 1.0120x over previous
import jax
import jax.numpy as jnp
from jax.experimental import pallas as pl
from jax.experimental.pallas import tpu as pltpu

_K = 64
_N = 12500
_D = 128
_NB = _N // _D
_TAIL = _N - _NB * _D
_MROWS = 128
_BPB = 4
_NEG = float("-inf")


def _roll_lanes(x, s):
    return jnp.concatenate([x[..., -s:], x[..., :-s]], axis=-1)


def _lane_iota(shape):
    return jax.lax.broadcasted_iota(jnp.int32, shape, len(shape) - 1)


def _kv_stage(keys, idx, j, take_min):
    bit = (_lane_iota(keys.shape) & j) != 0
    kp = jnp.where(bit, _roll_lanes(keys, j), _roll_lanes(keys, -j))
    ip = jnp.where(bit, _roll_lanes(idx, j), _roll_lanes(idx, -j))
    self_hi = (keys > kp) | ((keys == kp) & (idx < ip))
    keep = self_hi != take_min
    return jnp.where(keep, keys, kp), jnp.where(keep, idx, ip)


def _sort_rows(keys, idx, desc_mask):
    lane = _lane_iota(keys.shape)
    k2 = 2
    while k2 <= _D:
        j = k2 // 2
        while j >= 1:
            take_min_asc = ((lane & k2) == 0) == ((lane & j) == 0)
            keys, idx = _kv_stage(keys, idx, j, take_min_asc != desc_mask)
            j //= 2
        k2 *= 2
    return keys, idx


def _merge_rows(keys, idx, desc_mask):
    lane = _lane_iota(keys.shape)
    j = _D // 2
    while j >= 1:
        keys, idx = _kv_stage(keys, idx, j, ((lane & j) == 0) != desc_mask)
        j //= 2
    return keys, idx


def _row_dir_mask(rows):
    if rows == 1:
        return jnp.full((1, _D), True)
    sub = jax.lax.broadcasted_iota(jnp.int32, (rows, _D), 0)
    return sub >= rows // 2


def _sort_rows_asc_plain(x):
    lane = _lane_iota(x.shape)
    k2 = 2
    while k2 <= _D:
        j = k2 // 2
        while j >= 1:
            bit = (lane & j) != 0
            p = jnp.where(bit, _roll_lanes(x, j), _roll_lanes(x, -j))
            take_min = ((lane & k2) == 0) == ((lane & j) == 0)
            x = jnp.where(take_min, jnp.minimum(x, p), jnp.maximum(x, p))
            j //= 2
        k2 *= 2
    return x


def _batch_keys(feat_ref, bb):
    mrows = []
    for b in range(_NB):
        t = jnp.transpose(feat_ref[bb, b * _D : (b + 1) * _D, :])
        mrows.append(jnp.max(t, axis=0, keepdims=True))
    tail = feat_ref[bb, _NB * _D : _N, :]
    mt = jnp.transpose(jnp.max(tail, axis=1, keepdims=True))
    mrows.append(jnp.concatenate(
        [mt, jnp.full((1, _D - _TAIL), _NEG, jnp.float32)], axis=1))
    return jnp.concatenate(mrows, axis=0)


def _tourney_round(keys, idx, h):
    ka, ia = keys[:, :h], idx[:, :h]
    kb, ib = keys[:, h:], idx[:, h:]
    self_hi = (ka > kb) | ((ka == kb) & (ia < ib))
    keys = jnp.where(self_hi, ka, kb)
    idx = jnp.where(self_hi, ia, ib)
    return _merge_rows(keys, idx, _row_dir_mask(h))


def _pair_topk(feat_ref, bb0, npair):
    nreal = _NB + 1
    keys = jnp.stack([_batch_keys(feat_ref, bb0 + p) for p in range(npair)])
    sub = jax.lax.broadcasted_iota(jnp.int32, keys.shape, 1)
    idx = sub * _D + _lane_iota(keys.shape)
    sortdir = jax.lax.broadcasted_iota(
        jnp.int32, (nreal, _D), 0) >= _MROWS // 2
    keys, idx = _sort_rows(keys, idx, sortdir)
    pad_k = jnp.full((npair, _MROWS - nreal, _D), _NEG, jnp.float32)
    pad_i = jnp.full((npair, _MROWS - nreal, _D), jnp.int32(1 << 20))
    keys = jnp.concatenate([keys, pad_k], axis=1)
    idx = jnp.concatenate([idx, pad_i], axis=1)
    return _tourney_round(keys, idx, _MROWS // 2)


def _sortpool_kernel(feat_ref, out_ref, sel_ref):
    parts = [_pair_topk(feat_ref, bb0, 2) for bb0 in range(0, _BPB, 2)]
    keys = jnp.concatenate([p[0] for p in parts], axis=0)
    idx = jnp.concatenate([p[1] for p in parts], axis=0)
    r = _MROWS // 2
    while r > 1:
        keys, idx = _tourney_round(keys, idx, r // 2)
        r //= 2

    for k in range(_K):
        for bb in range(_BPB):
            s = idx[bb, 0, k]
            sel_ref[bb, k : k + 1, :] = feat_ref[bb, pl.ds(s, 1), :]
    out_ref[...] = _sort_rows_asc_plain(sel_ref[...])


def kernel(feat):
    b = feat.shape[0]
    pooled = pl.pallas_call(
        _sortpool_kernel,
        grid=(b // _BPB,),
        in_specs=[pl.BlockSpec((_BPB, _N, _D), lambda i: (i, 0, 0))],
        out_specs=pl.BlockSpec((_BPB, _K, _D), lambda i: (i, 0, 0)),
        out_shape=jax.ShapeDtypeStruct((b, _K, _D), feat.dtype),
        scratch_shapes=[pltpu.VMEM((_BPB, _K, _D), jnp.float32)],
    )(feat)
    return pooled.reshape(b, _K * _D)


# speedup vs baseline: 267.6051x; 1.0001x over previous
import jax
import jax.numpy as jnp
from jax.experimental import pallas as pl
from jax.experimental.pallas import tpu as pltpu

_K = 64
_N = 12500
_D = 128
_NB = _N // _D
_TAIL = _N - _NB * _D
_MROWS = 128
_BPB = 4
_NEG = float("-inf")


def _roll_lanes(x, s):
    return pltpu.roll(x, s % x.shape[-1], x.ndim - 1)


def _lane_iota(shape):
    return jax.lax.broadcasted_iota(jnp.int32, shape, len(shape) - 1)


def _kv_stage(keys, idx, j, take_min):
    bit = (_lane_iota(keys.shape) & j) != 0
    kp = jnp.where(bit, _roll_lanes(keys, j), _roll_lanes(keys, -j))
    ip = jnp.where(bit, _roll_lanes(idx, j), _roll_lanes(idx, -j))
    self_hi = (keys > kp) | ((keys == kp) & (idx < ip))
    keep = self_hi != take_min
    return jnp.where(keep, keys, kp), jnp.where(keep, idx, ip)


def _sort_rows(keys, idx, desc_mask):
    lane = _lane_iota(keys.shape)
    k2 = 2
    while k2 <= _D:
        j = k2 // 2
        while j >= 1:
            take_min_asc = ((lane & k2) == 0) == ((lane & j) == 0)
            keys, idx = _kv_stage(keys, idx, j, take_min_asc != desc_mask)
            j //= 2
        k2 *= 2
    return keys, idx


def _merge_rows(keys, idx, desc_mask):
    lane = _lane_iota(keys.shape)
    j = _D // 2
    while j >= 1:
        keys, idx = _kv_stage(keys, idx, j, ((lane & j) == 0) != desc_mask)
        j //= 2
    return keys, idx


def _row_dir_mask(rows):
    if rows == 1:
        return jnp.full((1, _D), True)
    sub = jax.lax.broadcasted_iota(jnp.int32, (rows, _D), 0)
    return sub >= rows // 2


def _sort_rows_asc_plain(x):
    lane = _lane_iota(x.shape)
    k2 = 2
    while k2 <= _D:
        j = k2 // 2
        while j >= 1:
            bit = (lane & j) != 0
            p = jnp.where(bit, _roll_lanes(x, j), _roll_lanes(x, -j))
            take_min = ((lane & k2) == 0) == ((lane & j) == 0)
            x = jnp.where(take_min, jnp.minimum(x, p), jnp.maximum(x, p))
            j //= 2
        k2 *= 2
    return x


def _batch_keys(feat_ref, bb):
    mrows = []
    for b in range(_NB):
        t = jnp.transpose(feat_ref[bb, b * _D : (b + 1) * _D, :])
        mrows.append(jnp.max(t, axis=0, keepdims=True))
    tail = feat_ref[bb, _NB * _D : _N, :]
    mt = jnp.transpose(jnp.max(tail, axis=1, keepdims=True))
    mrows.append(jnp.concatenate(
        [mt, jnp.full((1, _D - _TAIL), _NEG, jnp.float32)], axis=1))
    return jnp.concatenate(mrows, axis=0)


def _tourney_round(keys, idx, h):
    ka, ia = keys[:, :h], idx[:, :h]
    kb, ib = keys[:, h:], idx[:, h:]
    self_hi = (ka > kb) | ((ka == kb) & (ia < ib))
    keys = jnp.where(self_hi, ka, kb)
    idx = jnp.where(self_hi, ia, ib)
    return _merge_rows(keys, idx, _row_dir_mask(h))


def _pair_topk(feat_ref, bb0, npair):
    nreal = _NB + 1
    keys = jnp.stack([_batch_keys(feat_ref, bb0 + p) for p in range(npair)])
    sub = jax.lax.broadcasted_iota(jnp.int32, keys.shape, 1)
    idx = sub * _D + _lane_iota(keys.shape)
    sortdir = jax.lax.broadcasted_iota(
        jnp.int32, (nreal, _D), 0) >= _MROWS // 2
    keys, idx = _sort_rows(keys, idx, sortdir)
    pad_k = jnp.full((npair, _MROWS - nreal, _D), _NEG, jnp.float32)
    pad_i = jnp.full((npair, _MROWS - nreal, _D), jnp.int32(1 << 20))
    keys = jnp.concatenate([keys, pad_k], axis=1)
    idx = jnp.concatenate([idx, pad_i], axis=1)
    return _tourney_round(keys, idx, _MROWS // 2)


def _sortpool_kernel(feat_ref, out_ref, sel_ref):
    parts = [_pair_topk(feat_ref, bb0, 2) for bb0 in range(0, _BPB, 2)]
    keys = jnp.concatenate([p[0] for p in parts], axis=0)
    idx = jnp.concatenate([p[1] for p in parts], axis=0)
    r = _MROWS // 2
    while r > 1:
        keys, idx = _tourney_round(keys, idx, r // 2)
        r //= 2

    scalars = [(bb, k, idx[bb, 0, k])
               for k in range(_K) for bb in range(_BPB)]
    for bb, k, s in scalars:
        sel_ref[bb, k : k + 1, :] = feat_ref[bb, pl.ds(s, 1), :]
    out_ref[...] = _sort_rows_asc_plain(sel_ref[...])


def kernel(feat):
    b = feat.shape[0]
    pooled = pl.pallas_call(
        _sortpool_kernel,
        grid=(b // _BPB,),
        in_specs=[pl.BlockSpec((_BPB, _N, _D), lambda i: (i, 0, 0))],
        out_specs=pl.BlockSpec((_BPB, _K, _D), lambda i: (i, 0, 0)),
        out_shape=jax.ShapeDtypeStruct((b, _K, _D), feat.dtype),
        scratch_shapes=[pltpu.VMEM((_BPB, _K, _D), jnp.float32)],
    )(feat)
    return pooled.reshape(b, _K * _D)
